# baseline (speedup 1.0000x reference)
import dataclasses
import functools

import jax
import jax.numpy as jnp
from jax import lax
from jax.experimental import pallas as pl
from jax.experimental.pallas import tpu as pltpu
from jax.experimental.pallas import tpu_sc as plsc

N_NODES = 10000
N_PAD = 10112
IN_CH = 128
OUT_CH = 128
NUM_T = 7
SCALING = 100.0
WIDTH = 144
NC = 2
NS = 16
NW = NC * NS
CHUNK = 128
ROW_BLK = 1000


def _scaled_table(x_src, weight, relation_weight):
    def body(rw_ref, x_ref, w_ref, out_ref, acc_ref):
        t = pl.program_id(1)

        @pl.when(t == 0)
        def _():
            acc_ref[...] = jnp.dot(
                x_ref[...], w_ref[...], preferred_element_type=jnp.float32
            )

        s = rw_ref[t] * SCALING
        s = jnp.where(s >= 0.0, s, 0.01 * s)
        out_ref[:, :OUT_CH] = acc_ref[...] * s
        out_ref[:, OUT_CH:] = jnp.full((ROW_BLK, WIDTH - OUT_CH), s, jnp.float32)

    n_blk = N_NODES // ROW_BLK
    return pl.pallas_call(
        body,
        grid=(n_blk, NUM_T),
        in_specs=[
            pl.BlockSpec(memory_space=pltpu.SMEM),
            pl.BlockSpec((ROW_BLK, IN_CH), lambda i, t: (i, 0)),
            pl.BlockSpec((IN_CH, OUT_CH), lambda i, t: (0, 0)),
        ],
        out_specs=pl.BlockSpec((ROW_BLK, WIDTH), lambda i, t: (t * n_blk + i, 0)),
        out_shape=jax.ShapeDtypeStruct((NUM_T * N_NODES, WIDTH), jnp.float32),
        scratch_shapes=[pltpu.VMEM((ROW_BLK, OUT_CH), jnp.float32)],
    )(relation_weight, x_src, weight)


CHUNKS_PER_TILE = 80


def _sc_aggregate(table, packed, zeros):
    rows_per_sub = N_PAD // NS
    nchunks = CHUNKS_PER_TILE
    mesh = plsc.VectorSubcoreMesh(core_axis_name="c", subcore_axis_name="s")

    @functools.partial(
        pl.kernel,
        mesh=mesh,
        out_type=jax.ShapeDtypeStruct((NC, N_PAD, WIDTH), jnp.float32),
        scratch_types=[
            [pltpu.VMEM((3, CHUNK), jnp.int32) for _ in range(4)],
            [pltpu.VMEM((CHUNK,), jnp.int32) for _ in range(2)],
            [pltpu.VMEM((CHUNK, WIDTH), jnp.float32) for _ in range(2)],
            pltpu.VMEM_SHARED((N_PAD, WIDTH), jnp.float32),
            [pltpu.SemaphoreType.DMA for _ in range(4)],
            [pltpu.SemaphoreType.DMA for _ in range(2)],
        ],
        compiler_params=dataclasses.replace(
            pltpu.CompilerParams(), use_tc_tiling_on_sc=False
        ),
    )
    def k(table_hbm, idx_hbm, zeros_hbm, out_hbm,
          idx_r, gidx, rows, acc, isem, gsem):
        cid = lax.axis_index("c")
        sid = lax.axis_index("s")
        wid = sid * NC + cid

        sub_slc = pl.ds(sid * rows_per_sub, rows_per_sub)
        pltpu.sync_copy(zeros_hbm.at[sub_slc], acc.at[sub_slc])
        plsc.subcore_barrier()

        def idx_load(j, q):
            pltpu.async_copy(idx_hbm.at[wid, j], idx_r[q], isem[q])

        def idx_wait(q):
            pltpu.make_async_copy(idx_hbm.at[wid, 0], idx_r[q], isem[q]).wait()

        def compute_gidx(q, x):
            @pl.loop(0, CHUNK // 16)
            def _(kk):
                sl = pl.ds(kk * 16, 16)
                gidx[x][sl] = idx_r[q][1, sl] * N_NODES + idx_r[q][0, sl]

        def gather_start(x):
            pltpu.async_copy(table_hbm.at[gidx[x]], rows[x], gsem[x])

        def gather_wait(x):
            pltpu.make_async_copy(table_hbm.at[gidx[x]], rows[x], gsem[x]).wait()

        def scatter(q, x):
            pltpu.sync_copy(rows[x], acc.at[idx_r[q].at[2]], add=True)

        for q in range(4):
            idx_load(q, q)
        idx_wait(0)
        compute_gidx(0, 0)
        gather_start(0)

        @pl.loop(0, nchunks // 4 - 1)
        def _(m):
            j0 = 4 * m
            for r in range(4):
                j = j0 + r
                x, xn, q, qn = r % 2, (r + 1) % 2, r, (r + 1) % 4
                idx_wait(qn)
                compute_gidx(qn, xn)
                gather_start(xn)
                gather_wait(x)
                scatter(q, x)
                idx_load(j + 4, q)

        for j in range(nchunks - 4, nchunks):
            r = j % 4
            x, xn, q, qn = r % 2, (r + 1) % 2, r, (r + 1) % 4
            if j + 1 < nchunks:
                idx_wait(qn)
                compute_gidx(qn, xn)
                gather_start(xn)
            gather_wait(x)
            scatter(q, x)

        plsc.subcore_barrier()
        pltpu.sync_copy(acc.at[sub_slc], out_hbm.at[cid, sub_slc])

    return k(table, packed, zeros)


def _finalize(partial, bias):
    def body(p_ref, b_ref, o_ref):
        a = p_ref[0] + p_ref[1]
        deg = a[:, OUT_CH:OUT_CH + 1]
        inv = jnp.where(deg != 0.0, 1.0 / deg, 0.0)
        o_ref[...] = a[:, :OUT_CH] * inv + b_ref[...]

    n_blk = N_NODES // ROW_BLK
    return pl.pallas_call(
        body,
        grid=(n_blk,),
        in_specs=[
            pl.BlockSpec((NC, ROW_BLK, WIDTH), lambda i: (0, i, 0)),
            pl.BlockSpec((1, OUT_CH), lambda i: (0, 0)),
        ],
        out_specs=pl.BlockSpec((ROW_BLK, OUT_CH), lambda i: (i, 0)),
        out_shape=jax.ShapeDtypeStruct((N_NODES, OUT_CH), jnp.float32),
    )(partial, bias)


def kernel(x_src, x_target, edge_index, edge_type, target_node_type,
           weight, bias, relation_weight):
    n_edges = edge_index.shape[1]
    e_pad = NW * CHUNKS_PER_TILE * CHUNK
    pad = e_pad - n_edges
    assert pad >= 0
    row = edge_index[0].astype(jnp.int32)
    col = edge_index[1].astype(jnp.int32)
    ty = edge_type.astype(jnp.int32)
    row = jnp.concatenate([row, jnp.zeros((pad,), jnp.int32)])
    ty = jnp.concatenate([ty, jnp.zeros((pad,), jnp.int32)])
    col = jnp.concatenate([col, jnp.full((pad,), N_NODES, jnp.int32)])
    packed = jnp.stack([row, ty, col])
    packed = packed.reshape(3, NW, CHUNKS_PER_TILE, CHUNK).transpose(1, 2, 0, 3)
    table = _scaled_table(x_src, weight, relation_weight.astype(jnp.float32))
    zeros = jnp.zeros((N_PAD, WIDTH), jnp.float32)
    partial = _sc_aggregate(table, packed, zeros)
    return _finalize(partial, bias.reshape(1, OUT_CH))


# speedup vs baseline: 2.2398x; 2.2398x over previous
import dataclasses
import functools

import jax
import jax.numpy as jnp
from jax import lax
from jax.experimental import pallas as pl
from jax.experimental.pallas import tpu as pltpu
from jax.experimental.pallas import tpu_sc as plsc

N_NODES = 10000
N_PAD = 10112
IN_CH = 128
OUT_CH = 128
NUM_T = 7
SCALING = 100.0
WIDTH = 144
NC = 2
NS = 16
NW = NC * NS
CHUNK = 128
ROW_BLK = 1000


def _scaled_table(x_src, weight, relation_weight):
    def body(rw_ref, x_ref, w_ref, out_ref, acc_ref):
        t = pl.program_id(0)

        @pl.when(t == 0)
        def _():
            acc_ref[...] = jnp.dot(
                x_ref[...], w_ref[...], preferred_element_type=jnp.float32
            )

        s = rw_ref[t] * SCALING
        s = jnp.where(s >= 0.0, s, 0.01 * s)
        out_ref[:, :OUT_CH] = acc_ref[...] * s
        out_ref[:, OUT_CH:] = jnp.full((N_NODES, WIDTH - OUT_CH), s, jnp.float32)

    return pl.pallas_call(
        body,
        grid=(NUM_T,),
        in_specs=[
            pl.BlockSpec(memory_space=pltpu.SMEM),
            pl.BlockSpec((N_NODES, IN_CH), lambda t: (0, 0)),
            pl.BlockSpec((IN_CH, OUT_CH), lambda t: (0, 0)),
        ],
        out_specs=pl.BlockSpec((N_NODES, WIDTH), lambda t: (t, 0)),
        out_shape=jax.ShapeDtypeStruct((NUM_T * N_NODES, WIDTH), jnp.float32),
        scratch_shapes=[pltpu.VMEM((N_NODES, OUT_CH), jnp.float32)],
    )(relation_weight, x_src, weight)


EDGES_PER_TILE = 10000
N_FULL = EDGES_PER_TILE // CHUNK
TAIL = EDGES_PER_TILE - N_FULL * CHUNK
N_MAIN = (N_FULL // 4) * 4 - 4


def _sc_aggregate(table, edge_index, edge_type):
    rows_per_sub = N_PAD // NS
    mesh = plsc.VectorSubcoreMesh(core_axis_name="c", subcore_axis_name="s")

    @functools.partial(
        pl.kernel,
        mesh=mesh,
        out_type=[
            jax.ShapeDtypeStruct((N_PAD, WIDTH), jnp.float32),
            jax.ShapeDtypeStruct((N_PAD, WIDTH), jnp.float32),
        ],
        scratch_types=[
            [pltpu.VMEM((CHUNK,), jnp.int32) for _ in range(4)],
            [pltpu.VMEM((CHUNK,), jnp.int32) for _ in range(4)],
            [pltpu.VMEM((CHUNK,), jnp.int32) for _ in range(4)],
            [pltpu.VMEM((CHUNK,), jnp.int32) for _ in range(2)],
            [pltpu.VMEM((CHUNK, WIDTH), jnp.float32) for _ in range(2)],
            pltpu.VMEM((TAIL,), jnp.int32),
            pltpu.VMEM((TAIL,), jnp.int32),
            pltpu.VMEM_SHARED((N_PAD, WIDTH), jnp.float32),
            [pltpu.SemaphoreType.DMA for _ in range(4)],
            [pltpu.SemaphoreType.DMA for _ in range(2)],
        ],
        compiler_params=dataclasses.replace(
            pltpu.CompilerParams(), use_tc_tiling_on_sc=False
        ),
    )
    def k(table_hbm, ei_hbm, ty_hbm, out0_hbm, out1_hbm,
          row_r, ty_r, col_r, gidx, rows, col_t, gidx_t, acc, isem, gsem):
        cid = lax.axis_index("c")
        sid = lax.axis_index("s")
        wid = sid * NC + cid
        base = wid * EDGES_PER_TILE

        def idx_load(j, q):
            off = base + j * CHUNK
            pltpu.async_copy(ei_hbm.at[0, pl.ds(off, CHUNK)], row_r[q], isem[q])
            pltpu.async_copy(ei_hbm.at[1, pl.ds(off, CHUNK)], col_r[q], isem[q])
            pltpu.async_copy(ty_hbm.at[pl.ds(off, CHUNK)], ty_r[q], isem[q])

        def idx_wait(q):
            pltpu.make_async_copy(ei_hbm.at[0, pl.ds(0, CHUNK)], row_r[q], isem[q]).wait()
            pltpu.make_async_copy(ei_hbm.at[0, pl.ds(0, CHUNK)], col_r[q], isem[q]).wait()
            pltpu.make_async_copy(ty_hbm.at[pl.ds(0, CHUNK)], ty_r[q], isem[q]).wait()

        def compute_gidx(q, x):
            for kk in range(CHUNK // 16):
                sl = pl.ds(kk * 16, 16)
                gidx[x][sl] = ty_r[q][sl] * N_NODES + row_r[q][sl]

        def gather_start(x):
            pltpu.async_copy(table_hbm.at[gidx[x]], rows[x], gsem[x])

        def gather_wait(x):
            pltpu.make_async_copy(table_hbm.at[gidx[x]], rows[x], gsem[x]).wait()

        def scatter(q, x):
            pltpu.sync_copy(rows[x], acc.at[col_r[q]], add=True)

        for q in range(4):
            idx_load(q, q)

        @pl.loop(0, CHUNK)
        def _(r):
            for c in range(WIDTH // 16):
                rows[1][r, pl.ds(c * 16, 16)] = jnp.zeros((16,), jnp.float32)

        sub0 = sid * rows_per_sub
        for b in range(rows_per_sub // CHUNK):
            pltpu.sync_copy(rows[1], acc.at[pl.ds(sub0 + b * CHUNK, CHUNK)])
        rem = rows_per_sub % CHUNK
        pltpu.sync_copy(
            rows[1].at[pl.ds(0, rem)],
            acc.at[pl.ds(sub0 + (rows_per_sub // CHUNK) * CHUNK, rem)],
        )

        plsc.subcore_barrier()

        @pl.loop(0, N_MAIN // 4)
        def _(m):
            j0 = 4 * m
            for q in range(4):
                idx_wait(q)
                compute_gidx(q, 0)
                gather_start(0)
                gather_wait(0)
                scatter(q, 0)
                idx_load(j0 + q + 4, q)

        for j in range(N_MAIN, N_FULL):
            q = j % 4
            idx_wait(q)
            compute_gidx(q, 0)
            gather_start(0)
            gather_wait(0)
            scatter(q, 0)
            if j + 4 < N_FULL:
                idx_load(j + 4, q)

        toff = base + N_FULL * CHUNK
        pltpu.async_copy(ei_hbm.at[0, pl.ds(toff, TAIL)], row_r[0].at[pl.ds(0, TAIL)], isem[0])
        pltpu.async_copy(ei_hbm.at[1, pl.ds(toff, TAIL)], col_t, isem[1])
        pltpu.async_copy(ty_hbm.at[pl.ds(toff, TAIL)], ty_r[0].at[pl.ds(0, TAIL)], isem[2])
        pltpu.make_async_copy(ei_hbm.at[0, pl.ds(0, TAIL)], row_r[0].at[pl.ds(0, TAIL)], isem[0]).wait()
        pltpu.make_async_copy(ei_hbm.at[0, pl.ds(0, TAIL)], col_t, isem[1]).wait()
        pltpu.make_async_copy(ty_hbm.at[pl.ds(0, TAIL)], ty_r[0].at[pl.ds(0, TAIL)], isem[2]).wait()
        sl = pl.ds(0, TAIL)
        gidx_t[...] = ty_r[0][sl] * N_NODES + row_r[0][sl]
        pltpu.async_copy(table_hbm.at[gidx_t], rows[0].at[sl], gsem[0]).wait()
        pltpu.sync_copy(rows[0].at[sl], acc.at[col_t], add=True)

        plsc.subcore_barrier()
        sub_slc = pl.ds(sub0, rows_per_sub)

        @pl.when(cid == 0)
        def _():
            pltpu.sync_copy(acc.at[sub_slc], out0_hbm.at[sub_slc])

        @pl.when(cid == 1)
        def _():
            pltpu.sync_copy(acc.at[sub_slc], out1_hbm.at[sub_slc])

    return k(table, edge_index, edge_type)


def _finalize(p0, p1, bias):
    def body(p0_ref, p1_ref, b_ref, o_ref):
        a = p0_ref[...] + p1_ref[...]
        deg = a[:, OUT_CH:OUT_CH + 1]
        inv = jnp.where(deg != 0.0, 1.0 / deg, 0.0)
        o_ref[...] = a[:, :OUT_CH] * inv + b_ref[...]

    n_blk = N_NODES // ROW_BLK
    return pl.pallas_call(
        body,
        grid=(n_blk,),
        in_specs=[
            pl.BlockSpec((ROW_BLK, WIDTH), lambda i: (i, 0)),
            pl.BlockSpec((ROW_BLK, WIDTH), lambda i: (i, 0)),
            pl.BlockSpec((1, OUT_CH), lambda i: (0, 0)),
        ],
        out_specs=pl.BlockSpec((ROW_BLK, OUT_CH), lambda i: (i, 0)),
        out_shape=jax.ShapeDtypeStruct((N_NODES, OUT_CH), jnp.float32),
    )(p0, p1, bias)


def kernel(x_src, x_target, edge_index, edge_type, target_node_type,
           weight, bias, relation_weight):
    ei = edge_index.astype(jnp.int32)
    ty = edge_type.astype(jnp.int32)
    table = _scaled_table(x_src, weight, relation_weight.astype(jnp.float32))
    p0, p1 = _sc_aggregate(table, ei, ty)
    return _finalize(p0, p1, bias.reshape(1, OUT_CH))


# speedup vs baseline: 4.1638x; 1.8590x over previous
import dataclasses
import functools

import jax
import jax.numpy as jnp
from jax import lax
from jax.experimental import pallas as pl
from jax.experimental.pallas import tpu as pltpu
from jax.experimental.pallas import tpu_sc as plsc

N_NODES = 10000
N_PAD = 10112
IN_CH = 128
OUT_CH = 128
NUM_T = 7
SCALING = 100.0
NC = 2
NS = 16
NW = NC * NS
CHUNK = 128
ROW_BLK = 1000


def _scaled_table(x_src, weight, relation_weight):
    def body(rw_ref, x_ref, w_ref, out_ref, acc_ref):
        t = pl.program_id(0)

        @pl.when(t == 0)
        def _():
            acc_ref[...] = jnp.dot(
                x_ref[...], w_ref[...], preferred_element_type=jnp.float32
            )

        s = rw_ref[t] * SCALING
        s = jnp.where(s >= 0.0, s, 0.01 * s)
        out_ref[...] = acc_ref[...] * s

    return pl.pallas_call(
        body,
        grid=(NUM_T,),
        in_specs=[
            pl.BlockSpec(memory_space=pltpu.SMEM),
            pl.BlockSpec((N_NODES, IN_CH), lambda t: (0, 0)),
            pl.BlockSpec((IN_CH, OUT_CH), lambda t: (0, 0)),
        ],
        out_specs=pl.BlockSpec((N_NODES, OUT_CH), lambda t: (t, 0)),
        out_shape=jax.ShapeDtypeStruct((NUM_T * N_NODES, OUT_CH), jnp.float32),
        scratch_shapes=[pltpu.VMEM((N_NODES, OUT_CH), jnp.float32)],
    )(relation_weight, x_src, weight)


N_CHUNKS = 2500
N_UNIFORM = N_CHUNKS // NW
N_EXTRA = N_CHUNKS - N_UNIFORM * NW
N_MAIN = (N_UNIFORM // 4) * 4 - 4


def _sc_aggregate(table, row2, col2, ty2, w8):
    rows_per_sub = N_PAD // NS
    mesh = plsc.VectorSubcoreMesh(core_axis_name="c", subcore_axis_name="s")

    @functools.partial(
        pl.kernel,
        mesh=mesh,
        out_type=[
            jax.ShapeDtypeStruct((N_PAD, OUT_CH), jnp.float32),
            jax.ShapeDtypeStruct((N_PAD, OUT_CH), jnp.float32),
            jax.ShapeDtypeStruct((N_PAD, 16), jnp.float32),
            jax.ShapeDtypeStruct((N_PAD, 16), jnp.float32),
        ],
        scratch_types=[
            [pltpu.VMEM((CHUNK,), jnp.int32) for _ in range(4)],
            [pltpu.VMEM((CHUNK,), jnp.int32) for _ in range(4)],
            [pltpu.VMEM((CHUNK,), jnp.int32) for _ in range(4)],
            [pltpu.VMEM((CHUNK,), jnp.int32) for _ in range(2)],
            [pltpu.VMEM((CHUNK, OUT_CH), jnp.float32) for _ in range(2)],
            [pltpu.VMEM((CHUNK, 16), jnp.float32) for _ in range(2)],
            pltpu.VMEM((8,), jnp.float32),
            pltpu.VMEM_SHARED((N_PAD, OUT_CH), jnp.float32),
            pltpu.VMEM_SHARED((N_PAD, 16), jnp.float32),
            [pltpu.SemaphoreType.DMA for _ in range(4)],
            [pltpu.SemaphoreType.DMA for _ in range(2)],
        ],
        compiler_params=dataclasses.replace(
            pltpu.CompilerParams(),
            needs_layout_passes=False,
            use_tc_tiling_on_sc=False,
        ),
    )
    def k(table_hbm, row_hbm, col_hbm, ty_hbm, w8_hbm,
          out0_hbm, out1_hbm, outd0_hbm, outd1_hbm,
          row_r, ty_r, col_r, gidx, rows, wv, wtbl, acc, accd, isem, gsem):
        cid = lax.axis_index("c")
        sid = lax.axis_index("s")
        wid = sid * NC + cid

        def idx_load(j, q):
            ci = j * NW + wid
            pltpu.async_copy(row_hbm.at[ci], row_r[q], isem[q])
            pltpu.async_copy(col_hbm.at[ci], col_r[q], isem[q])
            pltpu.async_copy(ty_hbm.at[ci], ty_r[q], isem[q])

        def idx_wait(q):
            pltpu.make_async_copy(row_hbm.at[0], row_r[q], isem[q]).wait()
            pltpu.make_async_copy(row_hbm.at[0], col_r[q], isem[q]).wait()
            pltpu.make_async_copy(row_hbm.at[0], ty_r[q], isem[q]).wait()

        def compute_gidx_w(q, x):
            lane = lax.iota(jnp.int32, 16)
            zero16 = jnp.zeros((16,), jnp.int32)
            for kk in range(CHUNK // 16):
                sl = pl.ds(kk * 16, 16)
                t16 = ty_r[q][sl]
                gidx[x][sl] = t16 * N_NODES + row_r[q][sl]
                w16 = plsc.load_gather(wtbl, [t16])
                plsc.store_scatter(wv[x], [kk * 16 + lane, zero16], w16)

        def gather_start(x):
            pltpu.async_copy(table_hbm.at[gidx[x]], rows[x], gsem[x])

        def gather_wait(x):
            pltpu.make_async_copy(table_hbm.at[gidx[x]], rows[x], gsem[x]).wait()

        def scatter(q, x):
            pltpu.sync_copy(rows[x], acc.at[col_r[q]], add=True)
            pltpu.sync_copy(wv[x], accd.at[col_r[q]], add=True)

        for q in range(4):
            idx_load(q, q)
        pltpu.sync_copy(w8_hbm, wtbl)

        @pl.loop(0, CHUNK)
        def _(r):
            for c in range(OUT_CH // 16):
                rows[1][r, pl.ds(c * 16, 16)] = jnp.zeros((16,), jnp.float32)

        @pl.loop(0, CHUNK)
        def _(r):
            wv[0][r, pl.ds(0, 16)] = jnp.zeros((16,), jnp.float32)
            wv[1][r, pl.ds(0, 16)] = jnp.zeros((16,), jnp.float32)

        sub0 = sid * rows_per_sub
        n_full = rows_per_sub // CHUNK
        for b in range(n_full):
            pltpu.sync_copy(rows[1], acc.at[pl.ds(sub0 + b * CHUNK, CHUNK)])
            pltpu.sync_copy(wv[1], accd.at[pl.ds(sub0 + b * CHUNK, CHUNK)])
        rem = rows_per_sub % CHUNK
        pltpu.sync_copy(rows[1].at[pl.ds(0, rem)],
                        acc.at[pl.ds(sub0 + n_full * CHUNK, rem)])
        pltpu.sync_copy(wv[1].at[pl.ds(0, rem)],
                        accd.at[pl.ds(sub0 + n_full * CHUNK, rem)])

        idx_wait(0)
        compute_gidx_w(0, 0)
        gather_start(0)
        plsc.subcore_barrier()

        @pl.loop(0, N_MAIN // 4)
        def _(m):
            j0 = 4 * m
            for r in range(4):
                x, xn, q, qn = r % 2, (r + 1) % 2, r, (r + 1) % 4
                idx_wait(qn)
                compute_gidx_w(qn, xn)
                gather_start(xn)
                gather_wait(x)
                scatter(q, x)
                idx_load(j0 + r + 4, q)

        for j in range(N_MAIN, N_UNIFORM):
            x, xn, q, qn = j % 2, (j + 1) % 2, j % 4, (j + 1) % 4
            if j + 1 < N_UNIFORM:
                idx_wait(qn)
                compute_gidx_w(qn, xn)
                gather_start(xn)
            gather_wait(x)
            scatter(q, x)
            if j + 4 < N_UNIFORM:
                idx_load(j + 4, q)

        @pl.when(wid < N_EXTRA)
        def _():
            ci = N_UNIFORM * NW + wid
            pltpu.async_copy(row_hbm.at[ci], row_r[0], isem[0])
            pltpu.async_copy(col_hbm.at[ci], col_r[0], isem[0])
            pltpu.async_copy(ty_hbm.at[ci], ty_r[0], isem[0])
            idx_wait(0)
            compute_gidx_w(0, 0)
            pltpu.async_copy(table_hbm.at[gidx[0]], rows[0], gsem[0]).wait()
            scatter(0, 0)

        plsc.subcore_barrier()
        sub_slc = pl.ds(sub0, rows_per_sub)

        @pl.when(cid == 0)
        def _():
            pltpu.sync_copy(acc.at[sub_slc], out0_hbm.at[sub_slc])
            pltpu.sync_copy(accd.at[sub_slc], outd0_hbm.at[sub_slc])

        @pl.when(cid == 1)
        def _():
            pltpu.sync_copy(acc.at[sub_slc], out1_hbm.at[sub_slc])
            pltpu.sync_copy(accd.at[sub_slc], outd1_hbm.at[sub_slc])

    return k(table, row2, col2, ty2, w8)


def _finalize(p0, p1, d0, d1, bias):
    def body(p0_ref, p1_ref, d0_ref, d1_ref, b_ref, o_ref):
        a = p0_ref[...] + p1_ref[...]
        deg = d0_ref[:, 0:1] + d1_ref[:, 0:1]
        inv = jnp.where(deg != 0.0, 1.0 / deg, 0.0)
        o_ref[...] = a * inv + b_ref[...]

    n_blk = N_NODES // ROW_BLK
    return pl.pallas_call(
        body,
        grid=(n_blk,),
        in_specs=[
            pl.BlockSpec((ROW_BLK, OUT_CH), lambda i: (i, 0)),
            pl.BlockSpec((ROW_BLK, OUT_CH), lambda i: (i, 0)),
            pl.BlockSpec((ROW_BLK, 16), lambda i: (i, 0)),
            pl.BlockSpec((ROW_BLK, 16), lambda i: (i, 0)),
            pl.BlockSpec((1, OUT_CH), lambda i: (0, 0)),
        ],
        out_specs=pl.BlockSpec((ROW_BLK, OUT_CH), lambda i: (i, 0)),
        out_shape=jax.ShapeDtypeStruct((N_NODES, OUT_CH), jnp.float32),
    )(p0, p1, d0, d1, bias)


def kernel(x_src, x_target, edge_index, edge_type, target_node_type,
           weight, bias, relation_weight):
    ei = edge_index.astype(jnp.int32)
    row2 = ei[0].reshape(N_CHUNKS, CHUNK)
    col2 = ei[1].reshape(N_CHUNKS, CHUNK)
    ty2 = edge_type.astype(jnp.int32).reshape(N_CHUNKS, CHUNK)
    rw = relation_weight.astype(jnp.float32)
    w7 = rw * SCALING
    w7 = jnp.where(w7 >= 0.0, w7, 0.01 * w7)
    w8 = jnp.pad(w7, (0, 1))
    table = _scaled_table(x_src, weight, rw)
    p0, p1, d0, d1 = _sc_aggregate(table, row2, col2, ty2, w8)
    return _finalize(p0, p1, d0, d1, bias.reshape(1, OUT_CH))
